# baseline (speedup 1.0000x reference)
import functools

import jax
import jax.numpy as jnp
from jax import lax
from jax.experimental import pallas as pl
from jax.experimental.pallas import tpu as pltpu
from jax.experimental.pallas import tpu_sc as plsc

N_NODES = 10000
EMB = 128
K_SENT = N_NODES * N_NODES
KPAD = 100_007_936
RROWS = KPAD // 128
HALF = KPAD // 2
TRASH0 = K_SENT + 256

NE_ROWS = 2560
NL_ROWS = 128
NM_ROWS = 2560
TOTAL_PAD = 650240

_SC = plsc.VectorSubcoreMesh(core_axis_name="c", subcore_axis_name="s")


def _mm_body(x_ref, w_ref, o_ref):
    o_ref[...] = jax.lax.dot_general(
        x_ref[...], w_ref[...],
        dimension_numbers=(((1,), (1,)), ((), ())),
        preferred_element_type=jnp.float32,
    )


def _linear(mmsbm_val, W):
    M = mmsbm_val.shape[0]
    BM = 640
    return pl.pallas_call(
        _mm_body,
        grid=(M // BM,),
        in_specs=[
            pl.BlockSpec((BM, EMB), lambda i: (i, 0)),
            pl.BlockSpec((128, EMB), lambda i: (0, 0)),
        ],
        out_specs=pl.BlockSpec((BM, EMB), lambda i: (i, 0)),
        out_shape=jax.ShapeDtypeStruct((M, EMB), jnp.float32),
    )(mmsbm_val, W)


_ZB = 16384
_SHW = 262144
_HZ_T = HALF // 16


def _iota16():
    return lax.broadcasted_iota(jnp.int32, (16,), 0)


def _fill_words(buf, nwords, value):
    def body(i, _):
        buf[pl.ds(i * 16, 16)] = jnp.full((16,), value, jnp.int32)
        return 0
    lax.fori_loop(0, nwords // 16, body, 0)


def _scatter_ones(ke, kl, km, present, kbuf, ibuf, obuf, zbuf, zsh, sem):
    cid = lax.axis_index("c")
    sid = lax.axis_index("s")
    wid = cid * 16 + sid
    _fill_words(zbuf, _ZB, 0)
    pltpu.sync_copy(zbuf, zsh.at[pl.ds(sid * _ZB, _ZB)])
    _fill_words(obuf, 128, 1)
    plsc.subcore_barrier()

    stripe0 = cid * HALF + sid * _HZ_T
    n_full = _HZ_T // _SHW
    rem = _HZ_T - n_full * _SHW

    def zfire(i, _):
        pltpu.make_async_copy(
            zsh, present.at[pl.ds(stripe0 + i * _SHW, _SHW)], sem).start()
        return 0
    lax.fori_loop(0, n_full, zfire, 0)
    if rem:
        pltpu.make_async_copy(
            zsh.at[pl.ds(0, rem)],
            present.at[pl.ds(stripe0 + n_full * _SHW, rem)], sem).start()

    def zdrain(i, _):
        pltpu.make_async_copy(
            zsh, present.at[pl.ds(stripe0 + i * _SHW, _SHW)], sem).wait()
        return 0
    lax.fori_loop(0, n_full, zdrain, 0)
    if rem:
        pltpu.make_async_copy(
            zsh.at[pl.ds(0, rem)],
            present.at[pl.ds(stripe0 + n_full * _SHW, rem)], sem).wait()
    plsc.subcore_barrier()

    lo = cid * HALF
    hi = lo + HALF
    trash_base = TRASH0 + wid * 128

    def do_array(keys_hbm, nrows):
        rpt = nrows // 16
        base = sid * rpt
        pltpu.sync_copy(keys_hbm.at[pl.ds(base, rpt)], kbuf.at[pl.ds(0, rpt)])

        def chunk(j, _):
            for l in range(8):
                k = kbuf[j, pl.ds(l * 16, 16)]
                in_half = jnp.logical_and(k >= lo, k < hi)
                tr = trash_base + l * 16 + _iota16()
                ibuf[j, pl.ds(l * 16, 16)] = jnp.where(in_half, k, tr)
            pltpu.make_async_copy(obuf, present.at[ibuf.at[j]], sem).start()
            return 0
        lax.fori_loop(0, rpt, chunk, 0)

        def drain(j, _):
            pltpu.make_async_copy(obuf, present.at[ibuf.at[j]], sem).wait()
            return 0
        lax.fori_loop(0, rpt, drain, 0)

    do_array(ke, NE_ROWS)
    do_array(kl, NL_ROWS)
    do_array(km, NM_ROWS)


def _present_kernel(ke2, kl2, km2):
    f = pl.kernel(
        _scatter_ones,
        out_type=jax.ShapeDtypeStruct((KPAD,), jnp.int32),
        mesh=_SC,
        scratch_types=[
            pltpu.VMEM((NE_ROWS // 16, 128), jnp.int32),
            pltpu.VMEM((NE_ROWS // 16, 128), jnp.int32),
            pltpu.VMEM((128,), jnp.int32),
            pltpu.VMEM((_ZB,), jnp.int32),
            pltpu.VMEM_SHARED((_SHW,), jnp.int32),
            pltpu.SemaphoreType.DMA,
        ],
    )
    return f(ke2, kl2, km2)


_PB = 512


def _prefix_body(x_ref, o_ref, carry_ref):
    i = pl.program_id(0)

    @pl.when(i == 0)
    def _():
        carry_ref[0] = 0.0

    x = x_ref[...].astype(jnp.float32)
    r128 = lax.broadcasted_iota(jnp.int32, (128, 128), 0)
    c128 = lax.broadcasted_iota(jnp.int32, (128, 128), 1)
    upper = (r128 < c128).astype(jnp.float32)
    lane_pref = jax.lax.dot_general(
        x, upper, dimension_numbers=(((1,), (0,)), ((), ())),
        preferred_element_type=jnp.float32)
    ones_col = jnp.ones((128, 1), jnp.float32)
    rs = jax.lax.dot_general(
        x, ones_col, dimension_numbers=(((1,), (0,)), ((), ())),
        preferred_element_type=jnp.float32)
    rb = lax.broadcasted_iota(jnp.int32, (_PB, _PB), 0)
    cb = lax.broadcasted_iota(jnp.int32, (_PB, _PB), 1)
    lower = (rb > cb).astype(jnp.float32)
    row_pref = jax.lax.dot_general(
        lower, rs, dimension_numbers=(((1,), (0,)), ((), ())),
        preferred_element_type=jnp.float32)
    carry = carry_ref[0]
    o_ref[...] = (lane_pref + row_pref + carry).astype(jnp.int32)
    carry_ref[0] = carry + jnp.sum(rs)


def _prefix(present2d):
    return pl.pallas_call(
        _prefix_body,
        grid=(RROWS // _PB,),
        in_specs=[pl.BlockSpec((_PB, 128), lambda i: (i, 0))],
        out_specs=pl.BlockSpec((_PB, 128), lambda i: (i, 0)),
        out_shape=jax.ShapeDtypeStruct((RROWS, 128), jnp.int32),
        scratch_shapes=[pltpu.SMEM((1,), jnp.float32)],
    )(present2d)


def _rank_gather_body(P, ke, kl, km, re_, rl, rm, kbuf, rbuf, sem):
    cid = lax.axis_index("c")
    sid = lax.axis_index("s")
    wid = sid * 2 + cid

    def do_array(keys_hbm, ranks_hbm, nrows):
        rpt = nrows // 32
        base = wid * rpt
        pltpu.sync_copy(keys_hbm.at[pl.ds(base, rpt)], kbuf.at[pl.ds(0, rpt)])

        def chunk(j, _):
            pltpu.make_async_copy(P.at[kbuf.at[j]], rbuf.at[j], sem).start()
            return 0
        lax.fori_loop(0, rpt, chunk, 0)

        def drain(j, _):
            pltpu.make_async_copy(P.at[kbuf.at[j]], rbuf.at[j], sem).wait()
            return 0
        lax.fori_loop(0, rpt, drain, 0)
        pltpu.sync_copy(rbuf.at[pl.ds(0, rpt)], ranks_hbm.at[pl.ds(base, rpt)])

    do_array(ke, re_, NE_ROWS)
    do_array(kl, rl, NL_ROWS)
    do_array(km, rm, NM_ROWS)


def _rank_gather(P_flat, ke2, kl2, km2):
    f = pl.kernel(
        _rank_gather_body,
        out_type=(
            jax.ShapeDtypeStruct((NE_ROWS, 128), jnp.int32),
            jax.ShapeDtypeStruct((NL_ROWS, 128), jnp.int32),
            jax.ShapeDtypeStruct((NM_ROWS, 128), jnp.int32),
        ),
        mesh=_SC,
        scratch_types=[
            pltpu.VMEM((NE_ROWS // 32, 128), jnp.int32),
            pltpu.VMEM((NE_ROWS // 32, 128), jnp.int32),
            pltpu.SemaphoreType.DMA,
        ],
    )
    return f(P_flat, ke2, kl2, km2)


def kernel(mmsbm_index, mmsbm_val, edge_index, edge_attr, W, num_nodes):
    N = N_NODES
    mv = _linear(mmsbm_val, W)

    ar = jnp.arange(N, dtype=jnp.int32)
    ke = edge_index[0] * N + edge_index[1]
    kl = ar * (N + 1)
    km = mmsbm_index[0] * N + mmsbm_index[1]
    pad = lambda k, r: jnp.concatenate(
        [k, jnp.full((r * 128 - k.shape[0],), K_SENT, jnp.int32)]).reshape(r, 128)
    ke_p, kl_p, km_p = pad(ke, NE_ROWS), pad(kl, NL_ROWS), pad(km, NM_ROWS)

    present = _present_kernel(ke_p, kl_p, km_p)
    P = _prefix(present.reshape(RROWS, 128))
    re_, rl, rm = _rank_gather(P.reshape(-1), ke_p, kl_p, km_p)

    inv = jnp.concatenate(
        [re_.reshape(-1)[:320000], rl.reshape(-1)[:N], rm.reshape(-1)[:320000]])
    rows = jnp.concatenate([edge_index[0], ar, mmsbm_index[0]])
    cols = jnp.concatenate([edge_index[1], ar, mmsbm_index[1]])
    all_val = jnp.concatenate(
        [edge_attr, jnp.zeros((N, EMB), jnp.float32), mv], axis=0)
    out_val = jax.ops.segment_sum(all_val, inv, num_segments=TOTAL_PAD)
    rowbuf = jnp.full((TOTAL_PAD,), N, jnp.int32).at[inv].set(rows)
    colbuf = jnp.zeros((TOTAL_PAD,), jnp.int32).at[inv].set(cols)
    out_idx = jnp.stack([rowbuf[:650000], colbuf[:650000]])
    return out_idx, out_val[:650000]


# speedup vs baseline: 1.0142x; 1.0142x over previous
import functools

import jax
import jax.numpy as jnp
from jax import lax
from jax.experimental import pallas as pl
from jax.experimental.pallas import tpu as pltpu
from jax.experimental.pallas import tpu_sc as plsc

N_NODES = 10000
EMB = 128
K_SENT = N_NODES * N_NODES
KPAD = 100_007_936
RROWS = KPAD // 128
HALF = KPAD // 2
TRASH0 = K_SENT + 256

NE_ROWS = 2560
NL_ROWS = 128
NM_ROWS = 2560
TOTAL_PAD = 650240

_SC = plsc.VectorSubcoreMesh(core_axis_name="c", subcore_axis_name="s")


def _mm_body(x_ref, w_ref, o_ref):
    o_ref[...] = jax.lax.dot_general(
        x_ref[...], w_ref[...],
        dimension_numbers=(((1,), (1,)), ((), ())),
        preferred_element_type=jnp.float32,
    )


def _linear(mmsbm_val, W):
    M = mmsbm_val.shape[0]
    BM = 640
    return pl.pallas_call(
        _mm_body,
        grid=(M // BM,),
        in_specs=[
            pl.BlockSpec((BM, EMB), lambda i: (i, 0)),
            pl.BlockSpec((128, EMB), lambda i: (0, 0)),
        ],
        out_specs=pl.BlockSpec((BM, EMB), lambda i: (i, 0)),
        out_shape=jax.ShapeDtypeStruct((M, EMB), jnp.float32),
    )(mmsbm_val, W)


_ZB = 16384
_SHW = 262144
_HZ_T = HALF // 16


def _iota16():
    return lax.broadcasted_iota(jnp.int32, (16,), 0)


def _fill_words(buf, nwords, value):
    def body(i, _):
        buf[pl.ds(i * 16, 16)] = jnp.full((16,), value, jnp.int32)
        return 0
    lax.fori_loop(0, nwords // 16, body, 0)


def _scatter_ones(ke, kl, km, present, kbuf, ibuf, obuf, zbuf, zsh, sem):
    cid = lax.axis_index("c")
    sid = lax.axis_index("s")
    wid = cid * 16 + sid
    with jax.named_scope("zs_fill"):
        _fill_words(zbuf, _ZB, 0)
        pltpu.sync_copy(zbuf, zsh.at[pl.ds(sid * _ZB, _ZB)])
        _fill_words(obuf, 128, 1)
        plsc.subcore_barrier()

    stripe0 = cid * HALF + sid * _HZ_T
    n_full = _HZ_T // _SHW
    rem = _HZ_T - n_full * _SHW

    with jax.named_scope("zs_pump"):
        def zfire(i, _):
            pltpu.make_async_copy(
                zsh, present.at[pl.ds(stripe0 + i * _SHW, _SHW)], sem).start()
            return 0
        lax.fori_loop(0, n_full, zfire, 0)
        if rem:
            pltpu.make_async_copy(
                zsh.at[pl.ds(0, rem)],
                present.at[pl.ds(stripe0 + n_full * _SHW, rem)], sem).start()

        def zdrain(i, _):
            pltpu.make_async_copy(
                zsh, present.at[pl.ds(stripe0 + i * _SHW, _SHW)], sem).wait()
            return 0
        lax.fori_loop(0, n_full, zdrain, 0)
        if rem:
            pltpu.make_async_copy(
                zsh.at[pl.ds(0, rem)],
                present.at[pl.ds(stripe0 + n_full * _SHW, rem)], sem).wait()
        plsc.subcore_barrier()

    lo = cid * HALF
    hi = lo + HALF
    trash_base = TRASH0 + wid * 128

    def do_array(keys_hbm, nrows):
        rpt = nrows // 16
        base = sid * rpt
        pltpu.sync_copy(keys_hbm.at[pl.ds(base, rpt)], kbuf.at[pl.ds(0, rpt)])

        def chunk(j, _):
            for l in range(8):
                k = kbuf[j, pl.ds(l * 16, 16)]
                in_half = jnp.logical_and(k >= lo, k < hi)
                tr = trash_base + l * 16 + _iota16()
                ibuf[j, pl.ds(l * 16, 16)] = jnp.where(in_half, k, tr)
            pltpu.make_async_copy(obuf, present.at[ibuf.at[j]], sem).start()
            return 0
        lax.fori_loop(0, rpt, chunk, 0)

        def drain(j, _):
            pltpu.make_async_copy(obuf, present.at[ibuf.at[j]], sem).wait()
            return 0
        lax.fori_loop(0, rpt, drain, 0)

    with jax.named_scope("scat_e"):
        do_array(ke, NE_ROWS)
    with jax.named_scope("scat_l"):
        do_array(kl, NL_ROWS)
    with jax.named_scope("scat_m"):
        do_array(km, NM_ROWS)


def _present_kernel(ke2, kl2, km2):
    f = pl.kernel(
        _scatter_ones,
        out_type=jax.ShapeDtypeStruct((KPAD,), jnp.int32),
        mesh=_SC,
        scratch_types=[
            pltpu.VMEM((NE_ROWS // 16, 128), jnp.int32),
            pltpu.VMEM((NE_ROWS // 16, 128), jnp.int32),
            pltpu.VMEM((128,), jnp.int32),
            pltpu.VMEM((_ZB,), jnp.int32),
            pltpu.VMEM_SHARED((_SHW,), jnp.int32),
            pltpu.SemaphoreType.DMA,
        ],
    )
    return f(ke2, kl2, km2)


_PB = 512


def _prefix_body(x_ref, o_ref, carry_ref):
    i = pl.program_id(0)

    @pl.when(i == 0)
    def _():
        carry_ref[0] = 0.0

    x = x_ref[...].astype(jnp.float32)
    r128 = lax.broadcasted_iota(jnp.int32, (128, 128), 0)
    c128 = lax.broadcasted_iota(jnp.int32, (128, 128), 1)
    upper = (r128 < c128).astype(jnp.float32)
    lane_pref = jax.lax.dot_general(
        x, upper, dimension_numbers=(((1,), (0,)), ((), ())),
        preferred_element_type=jnp.float32)
    ones_col = jnp.ones((128, 1), jnp.float32)
    rs = jax.lax.dot_general(
        x, ones_col, dimension_numbers=(((1,), (0,)), ((), ())),
        preferred_element_type=jnp.float32)
    rb = lax.broadcasted_iota(jnp.int32, (_PB, _PB), 0)
    cb = lax.broadcasted_iota(jnp.int32, (_PB, _PB), 1)
    lower = (rb > cb).astype(jnp.float32)
    row_pref = jax.lax.dot_general(
        lower, rs, dimension_numbers=(((1,), (0,)), ((), ())),
        preferred_element_type=jnp.float32)
    carry = carry_ref[0]
    o_ref[...] = (lane_pref + row_pref + carry).astype(jnp.int32)
    carry_ref[0] = carry + jnp.sum(rs)


def _prefix(present2d):
    return pl.pallas_call(
        _prefix_body,
        grid=(RROWS // _PB,),
        in_specs=[pl.BlockSpec((_PB, 128), lambda i: (i, 0))],
        out_specs=pl.BlockSpec((_PB, 128), lambda i: (i, 0)),
        out_shape=jax.ShapeDtypeStruct((RROWS, 128), jnp.int32),
        scratch_shapes=[pltpu.SMEM((1,), jnp.float32)],
    )(present2d)


def _rank_gather_body(P, ke, kl, km, re_, rl, rm, kbuf, rbuf, sem):
    cid = lax.axis_index("c")
    sid = lax.axis_index("s")
    wid = sid * 2 + cid

    def do_array(keys_hbm, ranks_hbm, nrows):
        rpt = nrows // 32
        base = wid * rpt
        pltpu.sync_copy(keys_hbm.at[pl.ds(base, rpt)], kbuf.at[pl.ds(0, rpt)])

        def chunk(j, _):
            pltpu.make_async_copy(P.at[kbuf.at[j]], rbuf.at[j], sem).start()
            return 0
        lax.fori_loop(0, rpt, chunk, 0)

        def drain(j, _):
            pltpu.make_async_copy(P.at[kbuf.at[j]], rbuf.at[j], sem).wait()
            return 0
        lax.fori_loop(0, rpt, drain, 0)
        pltpu.sync_copy(rbuf.at[pl.ds(0, rpt)], ranks_hbm.at[pl.ds(base, rpt)])

    do_array(ke, re_, NE_ROWS)
    do_array(kl, rl, NL_ROWS)
    do_array(km, rm, NM_ROWS)


def _rank_gather(P_flat, ke2, kl2, km2):
    f = pl.kernel(
        _rank_gather_body,
        out_type=(
            jax.ShapeDtypeStruct((NE_ROWS, 128), jnp.int32),
            jax.ShapeDtypeStruct((NL_ROWS, 128), jnp.int32),
            jax.ShapeDtypeStruct((NM_ROWS, 128), jnp.int32),
        ),
        mesh=_SC,
        scratch_types=[
            pltpu.VMEM((NE_ROWS // 32, 128), jnp.int32),
            pltpu.VMEM((NE_ROWS // 32, 128), jnp.int32),
            pltpu.SemaphoreType.DMA,
        ],
    )
    return f(P_flat, ke2, kl2, km2)


def kernel(mmsbm_index, mmsbm_val, edge_index, edge_attr, W, num_nodes):
    N = N_NODES
    mv = _linear(mmsbm_val, W)

    ar = jnp.arange(N, dtype=jnp.int32)
    ke = edge_index[0] * N + edge_index[1]
    kl = ar * (N + 1)
    km = mmsbm_index[0] * N + mmsbm_index[1]
    pad = lambda k, r: jnp.concatenate(
        [k, jnp.full((r * 128 - k.shape[0],), K_SENT, jnp.int32)]).reshape(r, 128)
    ke_p, kl_p, km_p = pad(ke, NE_ROWS), pad(kl, NL_ROWS), pad(km, NM_ROWS)

    present = _present_kernel(ke_p, kl_p, km_p)
    P = _prefix(present.reshape(RROWS, 128))
    re_, rl, rm = _rank_gather(P.reshape(-1), ke_p, kl_p, km_p)

    inv = jnp.concatenate(
        [re_.reshape(-1)[:320000], rl.reshape(-1)[:N], rm.reshape(-1)[:320000]])
    rows = jnp.concatenate([edge_index[0], ar, mmsbm_index[0]])
    cols = jnp.concatenate([edge_index[1], ar, mmsbm_index[1]])
    all_val = jnp.concatenate(
        [edge_attr, jnp.zeros((N, EMB), jnp.float32), mv], axis=0)
    out_val = jax.ops.segment_sum(all_val, inv, num_segments=TOTAL_PAD)
    rowbuf = jnp.full((TOTAL_PAD,), N, jnp.int32).at[inv].set(rows)
    colbuf = jnp.zeros((TOTAL_PAD,), jnp.int32).at[inv].set(cols)
    out_idx = jnp.stack([rowbuf[:650000], colbuf[:650000]])
    return out_idx, out_val[:650000]


# speedup vs baseline: 1.0891x; 1.0738x over previous
import functools

import jax
import jax.numpy as jnp
from jax import lax
from jax.experimental import pallas as pl
from jax.experimental.pallas import tpu as pltpu
from jax.experimental.pallas import tpu_sc as plsc

N_NODES = 10000
EMB = 128
K_SENT = N_NODES * N_NODES
KPAD = 100_728_832
RROWS = KPAD // 128
HALF = KPAD // 2
TRASH_E = K_SENT + 128
TRASH_L = TRASH_E + 327_680
TRASH_M = TRASH_L + 16_384

NE_ROWS = 2560
NL_ROWS = 128
NM_ROWS = 2560
TOTAL_PAD = 650240

_SC = plsc.VectorSubcoreMesh(core_axis_name="c", subcore_axis_name="s")


def _mm_body(x_ref, w_ref, o_ref):
    o_ref[...] = jax.lax.dot_general(
        x_ref[...], w_ref[...],
        dimension_numbers=(((1,), (1,)), ((), ())),
        preferred_element_type=jnp.float32,
    )


def _linear(mmsbm_val, W):
    M = mmsbm_val.shape[0]
    BM = 640
    return pl.pallas_call(
        _mm_body,
        grid=(M // BM,),
        in_specs=[
            pl.BlockSpec((BM, EMB), lambda i: (i, 0)),
            pl.BlockSpec((128, EMB), lambda i: (0, 0)),
        ],
        out_specs=pl.BlockSpec((BM, EMB), lambda i: (i, 0)),
        out_shape=jax.ShapeDtypeStruct((M, EMB), jnp.float32),
    )(mmsbm_val, W)


_ZB = 16384
_SHW = 262144
_HZ_T = HALF // 16


def _iota16():
    return lax.broadcasted_iota(jnp.int32, (16,), 0)


def _fill_words(buf, nwords, value):
    def body(i, _):
        buf[pl.ds(i * 16, 16)] = jnp.full((16,), value, jnp.int32)
        return 0
    lax.fori_loop(0, nwords // 16, body, 0)


def _scatter_ones(ke, kl, km, present, kbuf, ibuf, obuf, zbuf, zsh, sem):
    cid = lax.axis_index("c")
    sid = lax.axis_index("s")
    wid = cid * 16 + sid
    with jax.named_scope("zs_fill"):
        _fill_words(zbuf, _ZB, 0)
        pltpu.sync_copy(zbuf, zsh.at[pl.ds(sid * _ZB, _ZB)])
        _fill_words(obuf, 128, 1)
        plsc.subcore_barrier()

    stripe0 = cid * HALF + sid * _HZ_T
    n_full = _HZ_T // _SHW
    rem = _HZ_T - n_full * _SHW

    with jax.named_scope("zs_pump"):
        def zfire(i, _):
            pltpu.make_async_copy(
                zsh, present.at[pl.ds(stripe0 + i * _SHW, _SHW)], sem).start()
            return 0
        lax.fori_loop(0, n_full, zfire, 0)
        if rem:
            pltpu.make_async_copy(
                zsh.at[pl.ds(0, rem)],
                present.at[pl.ds(stripe0 + n_full * _SHW, rem)], sem).start()

        def zdrain(i, _):
            pltpu.make_async_copy(
                zsh, present.at[pl.ds(stripe0 + i * _SHW, _SHW)], sem).wait()
            return 0
        lax.fori_loop(0, n_full, zdrain, 0)
        if rem:
            pltpu.make_async_copy(
                zsh.at[pl.ds(0, rem)],
                present.at[pl.ds(stripe0 + n_full * _SHW, rem)], sem).wait()
        plsc.subcore_barrier()

    lo = cid * HALF
    hi = lo + HALF

    def do_array(keys_hbm, nrows, trash_off):
        rpt = nrows // 16
        base = sid * rpt
        pltpu.sync_copy(keys_hbm.at[pl.ds(base, rpt)], kbuf.at[pl.ds(0, rpt)])

        def chunk(j, _):
            for l in range(8):
                k = kbuf[j, pl.ds(l * 16, 16)]
                in_half = jnp.logical_and(k >= lo, k < hi)
                tr = trash_off + (base + j) * 128 + l * 16 + _iota16()
                ibuf[j, pl.ds(l * 16, 16)] = jnp.where(in_half, k, tr)
            pltpu.make_async_copy(obuf, present.at[ibuf.at[j]], sem).start()
            return 0
        lax.fori_loop(0, rpt, chunk, 0)

        def drain(j, _):
            pltpu.make_async_copy(obuf, present.at[ibuf.at[j]], sem).wait()
            return 0
        lax.fori_loop(0, rpt, drain, 0)

    with jax.named_scope("scat_e"):
        do_array(ke, NE_ROWS, TRASH_E)
    with jax.named_scope("scat_l"):
        do_array(kl, NL_ROWS, TRASH_L)
    with jax.named_scope("scat_m"):
        do_array(km, NM_ROWS, TRASH_M)


def _present_kernel(ke2, kl2, km2):
    f = pl.kernel(
        _scatter_ones,
        out_type=jax.ShapeDtypeStruct((KPAD,), jnp.int32),
        mesh=_SC,
        scratch_types=[
            pltpu.VMEM((NE_ROWS // 16, 128), jnp.int32),
            pltpu.VMEM((NE_ROWS // 16, 128), jnp.int32),
            pltpu.VMEM((128,), jnp.int32),
            pltpu.VMEM((_ZB,), jnp.int32),
            pltpu.VMEM_SHARED((_SHW,), jnp.int32),
            pltpu.SemaphoreType.DMA,
        ],
    )
    return f(ke2, kl2, km2)


_PB = 512


def _prefix_body(x_ref, o_ref, carry_ref):
    i = pl.program_id(0)

    @pl.when(i == 0)
    def _():
        carry_ref[0] = 0.0

    x = x_ref[...].astype(jnp.float32)
    r128 = lax.broadcasted_iota(jnp.int32, (128, 128), 0)
    c128 = lax.broadcasted_iota(jnp.int32, (128, 128), 1)
    upper = (r128 < c128).astype(jnp.float32)
    lane_pref = jax.lax.dot_general(
        x, upper, dimension_numbers=(((1,), (0,)), ((), ())),
        preferred_element_type=jnp.float32)
    ones_col = jnp.ones((128, 1), jnp.float32)
    rs = jax.lax.dot_general(
        x, ones_col, dimension_numbers=(((1,), (0,)), ((), ())),
        preferred_element_type=jnp.float32)
    rb = lax.broadcasted_iota(jnp.int32, (_PB, _PB), 0)
    cb = lax.broadcasted_iota(jnp.int32, (_PB, _PB), 1)
    lower = (rb > cb).astype(jnp.float32)
    row_pref = jax.lax.dot_general(
        lower, rs, dimension_numbers=(((1,), (0,)), ((), ())),
        preferred_element_type=jnp.float32)
    carry = carry_ref[0]
    o_ref[...] = (lane_pref + row_pref + carry).astype(jnp.int32)
    carry_ref[0] = carry + jnp.sum(rs)


def _prefix(present2d):
    return pl.pallas_call(
        _prefix_body,
        grid=(RROWS // _PB,),
        in_specs=[pl.BlockSpec((_PB, 128), lambda i: (i, 0))],
        out_specs=pl.BlockSpec((_PB, 128), lambda i: (i, 0)),
        out_shape=jax.ShapeDtypeStruct((RROWS, 128), jnp.int32),
        scratch_shapes=[pltpu.SMEM((1,), jnp.float32)],
    )(present2d)


def _rank_gather_body(P, ke, kl, km, re_, rl, rm, kbuf, rbuf, sem):
    cid = lax.axis_index("c")
    sid = lax.axis_index("s")
    wid = sid * 2 + cid

    def do_array(keys_hbm, ranks_hbm, nrows):
        rpt = nrows // 32
        base = wid * rpt
        pltpu.sync_copy(keys_hbm.at[pl.ds(base, rpt)], kbuf.at[pl.ds(0, rpt)])

        def chunk(j, _):
            pltpu.make_async_copy(P.at[kbuf.at[j]], rbuf.at[j], sem).start()
            return 0
        lax.fori_loop(0, rpt, chunk, 0)

        def drain(j, _):
            pltpu.make_async_copy(P.at[kbuf.at[j]], rbuf.at[j], sem).wait()
            return 0
        lax.fori_loop(0, rpt, drain, 0)
        pltpu.sync_copy(rbuf.at[pl.ds(0, rpt)], ranks_hbm.at[pl.ds(base, rpt)])

    do_array(ke, re_, NE_ROWS)
    do_array(kl, rl, NL_ROWS)
    do_array(km, rm, NM_ROWS)


def _rank_gather(P_flat, ke2, kl2, km2):
    f = pl.kernel(
        _rank_gather_body,
        out_type=(
            jax.ShapeDtypeStruct((NE_ROWS, 128), jnp.int32),
            jax.ShapeDtypeStruct((NL_ROWS, 128), jnp.int32),
            jax.ShapeDtypeStruct((NM_ROWS, 128), jnp.int32),
        ),
        mesh=_SC,
        scratch_types=[
            pltpu.VMEM((NE_ROWS // 32, 128), jnp.int32),
            pltpu.VMEM((NE_ROWS // 32, 128), jnp.int32),
            pltpu.SemaphoreType.DMA,
        ],
    )
    return f(P_flat, ke2, kl2, km2)


def kernel(mmsbm_index, mmsbm_val, edge_index, edge_attr, W, num_nodes):
    N = N_NODES
    mv = _linear(mmsbm_val, W)

    ar = jnp.arange(N, dtype=jnp.int32)
    ke = edge_index[0] * N + edge_index[1]
    kl = ar * (N + 1)
    km = mmsbm_index[0] * N + mmsbm_index[1]
    pad = lambda k, r: jnp.concatenate(
        [k, jnp.full((r * 128 - k.shape[0],), K_SENT, jnp.int32)]).reshape(r, 128)
    ke_p, kl_p, km_p = pad(ke, NE_ROWS), pad(kl, NL_ROWS), pad(km, NM_ROWS)

    present = _present_kernel(ke_p, kl_p, km_p)
    P = _prefix(present.reshape(RROWS, 128))
    re_, rl, rm = _rank_gather(P.reshape(-1), ke_p, kl_p, km_p)

    inv = jnp.concatenate(
        [re_.reshape(-1)[:320000], rl.reshape(-1)[:N], rm.reshape(-1)[:320000]])
    rows = jnp.concatenate([edge_index[0], ar, mmsbm_index[0]])
    cols = jnp.concatenate([edge_index[1], ar, mmsbm_index[1]])
    all_val = jnp.concatenate(
        [edge_attr, jnp.zeros((N, EMB), jnp.float32), mv], axis=0)
    out_val = jax.ops.segment_sum(all_val, inv, num_segments=TOTAL_PAD)
    rowbuf = jnp.full((TOTAL_PAD,), N, jnp.int32).at[inv].set(rows)
    colbuf = jnp.zeros((TOTAL_PAD,), jnp.int32).at[inv].set(cols)
    out_idx = jnp.stack([rowbuf[:650000], colbuf[:650000]])
    return out_idx, out_val[:650000]
